# baseline (speedup 1.0000x reference)
import functools

import jax
import jax.numpy as jnp
from jax import lax
from jax.experimental import pallas as pl
from jax.experimental.pallas import tpu as pltpu
from jax.experimental.pallas import tpu_sc as plsc

N = 10000
E = 320000
D = 128
CLASSES = 20

NC = 2
NS = 16
K = 128
NCT = 80
EPC = NCT * K * NS
EH = E // NC
NTAB = 10400
NPA = 10240
NPT = NPA // NS
WB = 128
NB = 2
IBR = 8
J = 2 * IBR
NDB = NCT // J


def _pe_body(x_ref, pe_ref, o_ref):
    i = pl.program_id(0)

    @pl.when(i < 25)
    def _():
        o_ref[...] = x_ref[...] + pe_ref[...]

    @pl.when(i == 25)
    def _():
        o_ref[...] = jnp.zeros_like(o_ref)


def _sc_prop_body(h_hbm, src_hbm, dst_hbm, zeros_hbm, out_hbm,
                  s_b0, s_b1, d_b0, d_b1, rows_v, acc_sh,
                  gsem, ssem, isem):
    c = lax.axis_index("c")
    s = lax.axis_index("s")
    tile_row0 = (c * NS + s) * NCT
    nrow0 = s * NPT

    pltpu.sync_copy(src_hbm.at[pl.ds(tile_row0, IBR)], s_b0)
    pltpu.sync_copy(dst_hbm.at[pl.ds(tile_row0, IBR)], d_b0)
    pltpu.async_copy(src_hbm.at[pl.ds(tile_row0 + IBR, IBR)], s_b1, isem.at[1])
    pltpu.async_copy(dst_hbm.at[pl.ds(tile_row0 + IBR, IBR)], d_b1, isem.at[1])

    pltpu.sync_copy(zeros_hbm.at[pl.ds(nrow0, NPT)],
                    acc_sh.at[pl.ds(nrow0, NPT)])

    def wait_i(par, sb, db, r0):
        pltpu.make_async_copy(src_hbm.at[pl.ds(r0, IBR)], sb,
                              isem.at[par]).wait()
        pltpu.make_async_copy(dst_hbm.at[pl.ds(r0, IBR)], db,
                              isem.at[par]).wait()

    def fire_g(sb, r, b):
        pltpu.async_copy(h_hbm.at[sb.at[r]], rows_v.at[b], gsem.at[b])

    def wait_g(sb, r, b):
        pltpu.make_async_copy(h_hbm.at[sb.at[r]], rows_v.at[b],
                              gsem.at[b]).wait()

    def fire_s(db, r, b):
        pltpu.async_copy(rows_v.at[b], acc_sh.at[db.at[r]], ssem.at[b],
                         add=True)

    def wait_s(db, r, b):
        pltpu.make_async_copy(rows_v.at[b], acc_sh.at[db.at[r]],
                              ssem.at[b]).wait()

    fire_g(s_b0, 0, 0)
    fire_g(s_b0, 1, 1)
    plsc.subcore_barrier()

    sbufs = (s_b0, s_b1)
    dbufs = (d_b0, d_b1)

    def dblblock(t, carry):
        for j in range(J):
            b = j % NB
            par = j // IBR
            r = j % IBR
            if j == 0:
                @pl.when(t > 0)
                def _():
                    r0 = tile_row0 + (2 * t + 1) * IBR
                    pltpu.async_copy(src_hbm.at[pl.ds(r0, IBR)], s_b1,
                                     isem.at[1])
                    pltpu.async_copy(dst_hbm.at[pl.ds(r0, IBR)], d_b1,
                                     isem.at[1])
            if j == 6:
                wait_i(1, s_b1, d_b1, tile_row0 + (2 * t + 1) * IBR)
            if j == 8:
                @pl.when(t < NDB - 1)
                def _():
                    r0 = tile_row0 + (2 * t + 2) * IBR
                    pltpu.async_copy(src_hbm.at[pl.ds(r0, IBR)], s_b0,
                                     isem.at[0])
                    pltpu.async_copy(dst_hbm.at[pl.ds(r0, IBR)], d_b0,
                                     isem.at[0])
            wait_g(sbufs[par], r, b)
            fire_s(dbufs[par], r, b)
            wait_s(dbufs[par], r, b)
            jn = j + NB
            if jn < J:
                fire_g(sbufs[jn // IBR], jn % IBR, b)
            else:
                if j == J - 2:
                    @pl.when(t < NDB - 1)
                    def _():
                        wait_i(0, s_b0, d_b0, tile_row0 + (2 * t + 2) * IBR)
                fire_g(s_b0, jn - J, b)
        return carry

    lax.fori_loop(0, NDB, dblblock, 0)
    wait_g(s_b0, 0, 0)
    wait_g(s_b0, 1, 1)

    plsc.subcore_barrier()
    pltpu.sync_copy(acc_sh.at[pl.ds(nrow0, NPT)],
                    out_hbm.at[pl.ds(c * NPA + nrow0, NPT)])


def _post_body(p0_ref, p1_ref, x_ref, pe_ref, wmi_ref, wmo_ref, ws_ref,
               bi_ref, bo_ref, cin_ref, cout_ref, wdec_ref, bdec_ref,
               logp_ref, emb_ref):
    f32 = jnp.float32
    h = x_ref[...] + pe_ref[...]
    p = p0_ref[0] + p1_ref[0]
    wa = wmi_ref[...] + ws_ref[...]
    wb = wmo_ref[...] + ws_ref[...]
    dn = (((1,), (1,)), ((), ()))
    ic = lax.dot_general(p, wa, dn, precision=lax.Precision.HIGHEST,
                         preferred_element_type=f32) + bi_ref[...]
    oc = lax.dot_general(p, wb, dn, precision=lax.Precision.HIGHEST,
                         preferred_element_type=f32) + bo_ref[...]
    conv = cin_ref[...] * ic + cout_ref[...] * oc
    h2 = jnp.tanh(conv + h)
    logits = lax.dot_general(h2, wdec_ref[...], dn,
                             precision=lax.Precision.HIGHEST,
                             preferred_element_type=f32) + bdec_ref[...]
    m = jnp.max(logits, axis=1, keepdims=True)
    lse = jnp.log(jnp.sum(jnp.exp(logits - m), axis=1, keepdims=True)) + m
    logp_ref[...] = logits - lse
    nrm = jnp.sqrt(jnp.sum(h2 * h2, axis=1, keepdims=True))
    emb_ref[...] = h2 / (nrm + 1e-12)


def _run_sc_prop(h_pad, src2d, dst2d, zeros):
    f32 = jnp.float32
    sc_prop = functools.partial(
        pl.kernel,
        out_type=jax.ShapeDtypeStruct((NC * NPA, D), f32),
        mesh=plsc.VectorSubcoreMesh(core_axis_name="c", subcore_axis_name="s"),
        scratch_types=[
            pltpu.VMEM((IBR, K), jnp.int32),
            pltpu.VMEM((IBR, K), jnp.int32),
            pltpu.VMEM((IBR, K), jnp.int32),
            pltpu.VMEM((IBR, K), jnp.int32),
            pltpu.VMEM((NB, K, D), f32),
            pltpu.VMEM_SHARED((NPA, D), f32),
            pltpu.SemaphoreType.DMA((NB,)),
            pltpu.SemaphoreType.DMA((NB,)),
            pltpu.SemaphoreType.DMA((2,)),
        ],
    )(_sc_prop_body)
    return sc_prop(h_pad, src2d, dst2d, zeros)


_ROWS_BLK = 1000
_GRID = N // _ROWS_BLK


def kernel(x, pe_table, W_main_in, W_main_out, W_shared, b_main_in,
           b_main_out, b_shared_in, b_shared_out, C_in_vec, C_out_vec,
           W_dec, b_dec, edge_index):
    f32 = jnp.float32
    pe_flat = pe_table.reshape(1, D).astype(f32)

    h_pad = pl.pallas_call(
        _pe_body,
        grid=(NTAB // 400,),
        in_specs=[
            pl.BlockSpec((400, D), lambda i: (jnp.minimum(i, 24), 0)),
            pl.BlockSpec((1, D), lambda i: (0, 0)),
        ],
        out_specs=pl.BlockSpec((400, D), lambda i: (i, 0)),
        out_shape=jax.ShapeDtypeStruct((NTAB, D), f32),
    )(x, pe_flat)

    src = edge_index[0].astype(jnp.int32)
    dst = edge_index[1].astype(jnp.int32)
    pad = EPC - EH
    padsrc = jnp.full((pad,), N, jnp.int32)
    paddst = jnp.full((pad,), N, jnp.int32)
    src2d = jnp.concatenate([src[:EH], padsrc, src[EH:], padsrc]).reshape(-1, K)
    dst2d = jnp.concatenate([dst[:EH], paddst, dst[EH:], paddst]).reshape(-1, K)

    zeros = jnp.zeros((NPA, D), f32)
    p01 = _run_sc_prop(h_pad, src2d, dst2d, zeros).reshape(NC, NPA, D)

    cin = C_in_vec.astype(f32)
    cout = C_out_vec.astype(f32)
    row_spec = pl.BlockSpec((_ROWS_BLK, D), lambda i: (i, 0))
    w_spec = pl.BlockSpec((D, D), lambda i: (0, 0))
    b_spec = pl.BlockSpec((1, D), lambda i: (0, 0))
    c_spec = pl.BlockSpec((_ROWS_BLK, 1), lambda i: (i, 0))
    logp, emb = pl.pallas_call(
        _post_body,
        grid=(_GRID,),
        in_specs=[
            pl.BlockSpec((1, _ROWS_BLK, D), lambda i: (0, i, 0)),
            pl.BlockSpec((1, _ROWS_BLK, D), lambda i: (1, i, 0)),
            row_spec,
            b_spec,
            w_spec, w_spec, w_spec,
            b_spec, b_spec,
            c_spec, c_spec,
            pl.BlockSpec((CLASSES, D), lambda i: (0, 0)),
            pl.BlockSpec((1, CLASSES), lambda i: (0, 0)),
        ],
        out_specs=[
            pl.BlockSpec((_ROWS_BLK, CLASSES), lambda i: (i, 0)),
            pl.BlockSpec((_ROWS_BLK, D), lambda i: (i, 0)),
        ],
        out_shape=[
            jax.ShapeDtypeStruct((N, CLASSES), f32),
            jax.ShapeDtypeStruct((N, D), f32),
        ],
    )(p01, p01, x, pe_flat,
      W_main_in, W_main_out, W_shared,
      (b_main_in + b_shared_in).reshape(1, D),
      (b_main_out + b_shared_out).reshape(1, D),
      cin, cout, W_dec, b_dec.reshape(1, CLASSES))
    return (logp, emb)


# speedup vs baseline: 1.0020x; 1.0020x over previous
import functools

import jax
import jax.numpy as jnp
from jax import lax
from jax.experimental import pallas as pl
from jax.experimental.pallas import tpu as pltpu
from jax.experimental.pallas import tpu_sc as plsc

N = 10000
E = 320000
D = 128
CLASSES = 20

NC = 2
NS = 16
K = 128
NCT = 80
EPC = NCT * K * NS
EH = E // NC
NTAB = 10400
NPA = 10240
NPT = NPA // NS
WB = 128
NB = 2
IBR = 8
J = 2 * IBR
NDB = NCT // J


def _pe_body(x_ref, pe_ref, o_ref):
    i = pl.program_id(0)

    @pl.when(i < 25)
    def _():
        o_ref[...] = x_ref[...] + pe_ref[...]

    @pl.when(i == 25)
    def _():
        o_ref[...] = jnp.zeros_like(o_ref)


def _sc_prop_body(h_hbm, src_hbm, dst_hbm, zeros_hbm, out_hbm,
                  s_b0, s_b1, d_b0, d_b1, rows_v, acc_sh,
                  gsem, ssem, isem):
    c = lax.axis_index("c")
    s = lax.axis_index("s")
    tile_row0 = (c * NS + s) * NCT
    nrow0 = s * NPT

    pltpu.sync_copy(src_hbm.at[pl.ds(tile_row0, IBR)], s_b0)
    pltpu.sync_copy(dst_hbm.at[pl.ds(tile_row0, IBR)], d_b0)
    pltpu.async_copy(src_hbm.at[pl.ds(tile_row0 + IBR, IBR)], s_b1, isem.at[1])
    pltpu.async_copy(dst_hbm.at[pl.ds(tile_row0 + IBR, IBR)], d_b1, isem.at[1])

    pltpu.sync_copy(zeros_hbm.at[pl.ds(nrow0, NPT)],
                    acc_sh.at[pl.ds(nrow0, NPT)])

    def wait_i(par, sb, db, r0):
        pltpu.make_async_copy(src_hbm.at[pl.ds(r0, IBR)], sb,
                              isem.at[par]).wait()
        pltpu.make_async_copy(dst_hbm.at[pl.ds(r0, IBR)], db,
                              isem.at[par]).wait()

    def fire_g(sb, r, b):
        pltpu.async_copy(h_hbm.at[sb.at[r]], rows_v.at[b], gsem.at[b])

    def wait_g(sb, r, b):
        pltpu.make_async_copy(h_hbm.at[sb.at[r]], rows_v.at[b],
                              gsem.at[b]).wait()

    def fire_s(db, r, b):
        pltpu.async_copy(rows_v.at[b], acc_sh.at[db.at[r]], ssem.at[b],
                         add=True)

    def wait_s(db, r, b):
        pltpu.make_async_copy(rows_v.at[b], acc_sh.at[db.at[r]],
                              ssem.at[b]).wait()

    fire_g(s_b0, 0, 0)
    fire_g(s_b0, 1, 1)
    plsc.subcore_barrier()

    sbufs = (s_b0, s_b1)
    dbufs = (d_b0, d_b1)

    def dblblock(t, carry):
        for j in range(J):
            b = j % NB
            par = j // IBR
            r = j % IBR
            if j == 0:
                @pl.when(t > 0)
                def _():
                    r0 = tile_row0 + (2 * t + 1) * IBR
                    pltpu.async_copy(src_hbm.at[pl.ds(r0, IBR)], s_b1,
                                     isem.at[1])
                    pltpu.async_copy(dst_hbm.at[pl.ds(r0, IBR)], d_b1,
                                     isem.at[1])
            if j == 6:
                wait_i(1, s_b1, d_b1, tile_row0 + (2 * t + 1) * IBR)
            if j == 8:
                @pl.when(t < NDB - 1)
                def _():
                    r0 = tile_row0 + (2 * t + 2) * IBR
                    pltpu.async_copy(src_hbm.at[pl.ds(r0, IBR)], s_b0,
                                     isem.at[0])
                    pltpu.async_copy(dst_hbm.at[pl.ds(r0, IBR)], d_b0,
                                     isem.at[0])
            wait_g(sbufs[par], r, b)
            fire_s(dbufs[par], r, b)
            wait_s(dbufs[par], r, b)
            jn = j + NB
            if jn < J:
                fire_g(sbufs[jn // IBR], jn % IBR, b)
            else:
                if j == J - 2:
                    @pl.when(t < NDB - 1)
                    def _():
                        wait_i(0, s_b0, d_b0, tile_row0 + (2 * t + 2) * IBR)
                fire_g(s_b0, jn - J, b)
        return carry

    lax.fori_loop(0, NDB, dblblock, 0)
    wait_g(s_b0, 0, 0)
    wait_g(s_b0, 1, 1)

    plsc.subcore_barrier()
    out0 = c * NPA + nrow0
    for t in range(NPT // WB):
        pltpu.sync_copy(acc_sh.at[pl.ds(nrow0 + t * WB, WB)], rows_v.at[0])
        pltpu.sync_copy(rows_v.at[0], out_hbm.at[pl.ds(out0 + t * WB, WB)])


def _post_body(p0_ref, p1_ref, x_ref, pe_ref, wmi_ref, wmo_ref, ws_ref,
               bi_ref, bo_ref, cin_ref, cout_ref, wdec_ref, bdec_ref,
               logp_ref, emb_ref):
    f32 = jnp.float32
    h = x_ref[...] + pe_ref[...]
    p = p0_ref[0] + p1_ref[0]
    wa = wmi_ref[...] + ws_ref[...]
    wb = wmo_ref[...] + ws_ref[...]
    dn = (((1,), (1,)), ((), ()))
    ic = lax.dot_general(p, wa, dn, precision=lax.Precision.HIGHEST,
                         preferred_element_type=f32) + bi_ref[...]
    oc = lax.dot_general(p, wb, dn, precision=lax.Precision.HIGHEST,
                         preferred_element_type=f32) + bo_ref[...]
    conv = cin_ref[...] * ic + cout_ref[...] * oc
    h2 = jnp.tanh(conv + h)
    logits = lax.dot_general(h2, wdec_ref[...], dn,
                             precision=lax.Precision.HIGHEST,
                             preferred_element_type=f32) + bdec_ref[...]
    m = jnp.max(logits, axis=1, keepdims=True)
    lse = jnp.log(jnp.sum(jnp.exp(logits - m), axis=1, keepdims=True)) + m
    logp_ref[...] = logits - lse
    nrm = jnp.sqrt(jnp.sum(h2 * h2, axis=1, keepdims=True))
    emb_ref[...] = h2 / (nrm + 1e-12)


def _run_sc_prop(h_pad, src2d, dst2d, zeros):
    f32 = jnp.float32
    sc_prop = functools.partial(
        pl.kernel,
        out_type=jax.ShapeDtypeStruct((NC * NPA, D), f32),
        mesh=plsc.VectorSubcoreMesh(core_axis_name="c", subcore_axis_name="s"),
        scratch_types=[
            pltpu.VMEM((IBR, K), jnp.int32),
            pltpu.VMEM((IBR, K), jnp.int32),
            pltpu.VMEM((IBR, K), jnp.int32),
            pltpu.VMEM((IBR, K), jnp.int32),
            pltpu.VMEM((NB, K, D), f32),
            pltpu.VMEM_SHARED((NPA, D), f32),
            pltpu.SemaphoreType.DMA((NB,)),
            pltpu.SemaphoreType.DMA((NB,)),
            pltpu.SemaphoreType.DMA((2,)),
        ],
    )(_sc_prop_body)
    return sc_prop(h_pad, src2d, dst2d, zeros)


_ROWS_BLK = 1000
_GRID = N // _ROWS_BLK


def kernel(x, pe_table, W_main_in, W_main_out, W_shared, b_main_in,
           b_main_out, b_shared_in, b_shared_out, C_in_vec, C_out_vec,
           W_dec, b_dec, edge_index):
    f32 = jnp.float32
    pe_flat = pe_table.reshape(1, D).astype(f32)

    h_pad = pl.pallas_call(
        _pe_body,
        grid=(NTAB // 400,),
        in_specs=[
            pl.BlockSpec((400, D), lambda i: (jnp.minimum(i, 24), 0)),
            pl.BlockSpec((1, D), lambda i: (0, 0)),
        ],
        out_specs=pl.BlockSpec((400, D), lambda i: (i, 0)),
        out_shape=jax.ShapeDtypeStruct((NTAB, D), f32),
    )(x, pe_flat)

    src = edge_index[0].astype(jnp.int32)
    dst = edge_index[1].astype(jnp.int32)
    pad = EPC - EH
    padsrc = jnp.full((pad,), N, jnp.int32)
    paddst = jnp.full((pad,), N, jnp.int32)
    src2d = jnp.concatenate([src[:EH], padsrc, src[EH:], padsrc]).reshape(-1, K)
    dst2d = jnp.concatenate([dst[:EH], paddst, dst[EH:], paddst]).reshape(-1, K)

    zeros = jnp.zeros((NPA, D), f32)
    p01 = _run_sc_prop(h_pad, src2d, dst2d, zeros).reshape(NC, NPA, D)

    cin = C_in_vec.astype(f32)
    cout = C_out_vec.astype(f32)
    row_spec = pl.BlockSpec((_ROWS_BLK, D), lambda i: (i, 0))
    w_spec = pl.BlockSpec((D, D), lambda i: (0, 0))
    b_spec = pl.BlockSpec((1, D), lambda i: (0, 0))
    c_spec = pl.BlockSpec((_ROWS_BLK, 1), lambda i: (i, 0))
    logp, emb = pl.pallas_call(
        _post_body,
        grid=(_GRID,),
        in_specs=[
            pl.BlockSpec((1, _ROWS_BLK, D), lambda i: (0, i, 0)),
            pl.BlockSpec((1, _ROWS_BLK, D), lambda i: (1, i, 0)),
            row_spec,
            b_spec,
            w_spec, w_spec, w_spec,
            b_spec, b_spec,
            c_spec, c_spec,
            pl.BlockSpec((CLASSES, D), lambda i: (0, 0)),
            pl.BlockSpec((1, CLASSES), lambda i: (0, 0)),
        ],
        out_specs=[
            pl.BlockSpec((_ROWS_BLK, CLASSES), lambda i: (i, 0)),
            pl.BlockSpec((_ROWS_BLK, D), lambda i: (i, 0)),
        ],
        out_shape=[
            jax.ShapeDtypeStruct((N, CLASSES), f32),
            jax.ShapeDtypeStruct((N, D), f32),
        ],
    )(p01, p01, x, pe_flat,
      W_main_in, W_main_out, W_shared,
      (b_main_in + b_shared_in).reshape(1, D),
      (b_main_out + b_shared_out).reshape(1, D),
      cin, cout, W_dec, b_dec.reshape(1, CLASSES))
    return (logp, emb)


# speedup vs baseline: 1.0241x; 1.0220x over previous
import functools

import jax
import jax.numpy as jnp
from jax import lax
from jax.experimental import pallas as pl
from jax.experimental.pallas import tpu as pltpu
from jax.experimental.pallas import tpu_sc as plsc

N = 10000
E = 320000
D = 128
CLASSES = 20

NC = 2
NS = 16
K = 128
NCT = 80
EPC = NCT * K * NS
EH = E // NC
NTAB = 10400
NPA = 10240
NPT = NPA // NS
WB = 128
NB = 2
IBR = 16
NIB = NCT // IBR


def _pe_body(x_ref, pe_ref, o_ref):
    i = pl.program_id(0)

    @pl.when(i < 25)
    def _():
        o_ref[...] = x_ref[...] + pe_ref[...]

    @pl.when(i == 25)
    def _():
        o_ref[...] = jnp.zeros_like(o_ref)


def _sc_prop_body(h_hbm, src_hbm, dst_hbm, zeros_hbm, out_hbm,
                  src_b, dst_b, rows_v, acc_sh, gsem, ssem):
    c = lax.axis_index("c")
    s = lax.axis_index("s")
    tile_row0 = (c * NS + s) * NCT
    nrow0 = s * NPT

    pltpu.sync_copy(zeros_hbm.at[pl.ds(nrow0, NPT)], acc_sh.at[pl.ds(nrow0, NPT)])
    plsc.subcore_barrier()

    def fire_g(q, b):
        pltpu.async_copy(h_hbm.at[src_b.at[q]], rows_v.at[b], gsem.at[b])

    def wait_g(q, b):
        pltpu.make_async_copy(h_hbm.at[src_b.at[q]], rows_v.at[b],
                              gsem.at[b]).wait()

    def fire_s(q, b):
        pltpu.async_copy(rows_v.at[b], acc_sh.at[dst_b.at[q]], ssem.at[b],
                         add=True)

    def wait_s(q, b):
        pltpu.make_async_copy(rows_v.at[b], acc_sh.at[dst_b.at[q]],
                              ssem.at[b]).wait()

    def block(ib, carry):
        r0 = tile_row0 + ib * IBR
        pltpu.sync_copy(src_hbm.at[pl.ds(r0, IBR)], src_b)
        pltpu.sync_copy(dst_hbm.at[pl.ds(r0, IBR)], dst_b)
        for b in range(NB):
            fire_g(b, b)
        for q in range(IBR):
            b = q % NB
            wait_g(q, b)
            fire_s(q, b)
            if q + NB < IBR:
                wait_s(q, b)
                fire_g(q + NB, b)
        for q in range(IBR - NB, IBR):
            wait_s(q, q % NB)
        return carry

    lax.fori_loop(0, NIB, block, 0)
    plsc.subcore_barrier()

    out0 = c * NPA + nrow0

    def wb_block(j, carry):
        pltpu.sync_copy(acc_sh.at[pl.ds(nrow0 + j * WB, WB)], rows_v.at[0])
        pltpu.sync_copy(rows_v.at[0], out_hbm.at[pl.ds(out0 + j * WB, WB)])
        return carry

    lax.fori_loop(0, NPT // WB, wb_block, 0)


def _post_body(p0_ref, p1_ref, x_ref, pe_ref, wmi_ref, wmo_ref, ws_ref,
               bi_ref, bo_ref, cin_ref, cout_ref, wdec_ref, bdec_ref,
               logp_ref, emb_ref):
    f32 = jnp.float32
    h = x_ref[...] + pe_ref[...]
    p = p0_ref[0] + p1_ref[0]
    wa = wmi_ref[...] + ws_ref[...]
    wb = wmo_ref[...] + ws_ref[...]
    dn = (((1,), (1,)), ((), ()))
    ic = lax.dot_general(p, wa, dn, precision=lax.Precision.HIGHEST,
                         preferred_element_type=f32) + bi_ref[...]
    oc = lax.dot_general(p, wb, dn, precision=lax.Precision.HIGHEST,
                         preferred_element_type=f32) + bo_ref[...]
    conv = cin_ref[...] * ic + cout_ref[...] * oc
    h2 = jnp.tanh(conv + h)
    logits = lax.dot_general(h2, wdec_ref[...], dn,
                             precision=lax.Precision.HIGHEST,
                             preferred_element_type=f32) + bdec_ref[...]
    m = jnp.max(logits, axis=1, keepdims=True)
    lse = jnp.log(jnp.sum(jnp.exp(logits - m), axis=1, keepdims=True)) + m
    logp_ref[...] = logits - lse
    nrm = jnp.sqrt(jnp.sum(h2 * h2, axis=1, keepdims=True))
    emb_ref[...] = h2 / (nrm + 1e-12)


def _run_sc_prop(h_pad, src2d, dst2d, zeros):
    f32 = jnp.float32
    sc_prop = functools.partial(
        pl.kernel,
        out_type=jax.ShapeDtypeStruct((NC * NPA, D), f32),
        mesh=plsc.VectorSubcoreMesh(core_axis_name="c", subcore_axis_name="s"),
        scratch_types=[
            pltpu.VMEM((IBR, K), jnp.int32),
            pltpu.VMEM((IBR, K), jnp.int32),
            pltpu.VMEM((NB, K, D), f32),
            pltpu.VMEM_SHARED((NPA, D), f32),
            pltpu.SemaphoreType.DMA((NB,)),
            pltpu.SemaphoreType.DMA((NB,)),
        ],
    )(_sc_prop_body)
    return sc_prop(h_pad, src2d, dst2d, zeros)


_ROWS_BLK = 1000
_GRID = N // _ROWS_BLK


def kernel(x, pe_table, W_main_in, W_main_out, W_shared, b_main_in,
           b_main_out, b_shared_in, b_shared_out, C_in_vec, C_out_vec,
           W_dec, b_dec, edge_index):
    f32 = jnp.float32
    pe_flat = pe_table.reshape(1, D).astype(f32)

    h_pad = pl.pallas_call(
        _pe_body,
        grid=(NTAB // 400,),
        in_specs=[
            pl.BlockSpec((400, D), lambda i: (jnp.minimum(i, 24), 0)),
            pl.BlockSpec((1, D), lambda i: (0, 0)),
        ],
        out_specs=pl.BlockSpec((400, D), lambda i: (i, 0)),
        out_shape=jax.ShapeDtypeStruct((NTAB, D), f32),
    )(x, pe_flat)

    src = edge_index[0].astype(jnp.int32)
    dst = edge_index[1].astype(jnp.int32)
    pad = EPC - EH
    padsrc = jnp.full((pad,), N, jnp.int32)
    paddst = jnp.full((pad,), N, jnp.int32)
    src2d = jnp.concatenate([src[:EH], padsrc, src[EH:], padsrc]).reshape(-1, K)
    dst2d = jnp.concatenate([dst[:EH], paddst, dst[EH:], paddst]).reshape(-1, K)
    zeros = jnp.zeros((NPA, D), f32)

    p01 = _run_sc_prop(h_pad, src2d, dst2d, zeros).reshape(NC, NPA, D)

    cin = C_in_vec.astype(f32)
    cout = C_out_vec.astype(f32)
    row_spec = pl.BlockSpec((_ROWS_BLK, D), lambda i: (i, 0))
    w_spec = pl.BlockSpec((D, D), lambda i: (0, 0))
    b_spec = pl.BlockSpec((1, D), lambda i: (0, 0))
    c_spec = pl.BlockSpec((_ROWS_BLK, 1), lambda i: (i, 0))
    logp, emb = pl.pallas_call(
        _post_body,
        grid=(_GRID,),
        in_specs=[
            pl.BlockSpec((1, _ROWS_BLK, D), lambda i: (0, i, 0)),
            pl.BlockSpec((1, _ROWS_BLK, D), lambda i: (1, i, 0)),
            row_spec,
            b_spec,
            w_spec, w_spec, w_spec,
            b_spec, b_spec,
            c_spec, c_spec,
            pl.BlockSpec((CLASSES, D), lambda i: (0, 0)),
            pl.BlockSpec((1, CLASSES), lambda i: (0, 0)),
        ],
        out_specs=[
            pl.BlockSpec((_ROWS_BLK, CLASSES), lambda i: (i, 0)),
            pl.BlockSpec((_ROWS_BLK, D), lambda i: (i, 0)),
        ],
        out_shape=[
            jax.ShapeDtypeStruct((N, CLASSES), f32),
            jax.ShapeDtypeStruct((N, D), f32),
        ],
    )(p01, p01, x, pe_flat,
      W_main_in, W_main_out, W_shared,
      (b_main_in + b_shared_in).reshape(1, D),
      (b_main_out + b_shared_out).reshape(1, D),
      cin, cout, W_dec, b_dec.reshape(1, CLASSES))
    return (logp, emb)


# speedup vs baseline: 1.0298x; 1.0056x over previous
import functools

import jax
import jax.numpy as jnp
from jax import lax
from jax.experimental import pallas as pl
from jax.experimental.pallas import tpu as pltpu
from jax.experimental.pallas import tpu_sc as plsc

N = 10000
E = 320000
D = 128
CLASSES = 20

NC = 2
NS = 16
K = 128
NCT = 80
EPC = NCT * K * NS
EH = E // NC
NTAB = N + 8
NPA = 10240
NPT = NPA // NS
WB = 128
NB = 2
IBR = 16
NIB = NCT // IBR


def _pe_body(x_ref, pe_ref, o_ref):
    o_ref[...] = x_ref[...] + pe_ref[...]


def _sc_prop_body(h_hbm, src_hbm, dst_hbm, zeros_hbm, out_hbm,
                  src_b, dst_b, rows_v, acc_sh, gsem, ssem):
    c = lax.axis_index("c")
    s = lax.axis_index("s")
    tile_row0 = (c * NS + s) * NCT
    nrow0 = s * NPT

    pltpu.sync_copy(zeros_hbm.at[pl.ds(nrow0, NPT)], acc_sh.at[pl.ds(nrow0, NPT)])
    plsc.subcore_barrier()

    def fire_g(q, b):
        pltpu.async_copy(h_hbm.at[src_b.at[q]], rows_v.at[b], gsem.at[b])

    def wait_g(q, b):
        pltpu.make_async_copy(h_hbm.at[src_b.at[q]], rows_v.at[b],
                              gsem.at[b]).wait()

    def fire_s(q, b):
        pltpu.async_copy(rows_v.at[b], acc_sh.at[dst_b.at[q]], ssem.at[b],
                         add=True)

    def wait_s(q, b):
        pltpu.make_async_copy(rows_v.at[b], acc_sh.at[dst_b.at[q]],
                              ssem.at[b]).wait()

    def block(ib, carry):
        r0 = tile_row0 + ib * IBR
        pltpu.sync_copy(src_hbm.at[pl.ds(r0, IBR)], src_b)
        pltpu.sync_copy(dst_hbm.at[pl.ds(r0, IBR)], dst_b)
        for b in range(NB):
            fire_g(b, b)
        for q in range(IBR):
            b = q % NB
            wait_g(q, b)
            fire_s(q, b)
            if q + NB < IBR:
                wait_s(q, b)
                fire_g(q + NB, b)
        for q in range(IBR - NB, IBR):
            wait_s(q, q % NB)
        return carry

    lax.fori_loop(0, NIB, block, 0)
    plsc.subcore_barrier()

    out0 = c * NPA + nrow0

    def wb_block(j, carry):
        pltpu.sync_copy(acc_sh.at[pl.ds(nrow0 + j * WB, WB)], rows_v.at[0])
        pltpu.sync_copy(rows_v.at[0], out_hbm.at[pl.ds(out0 + j * WB, WB)])
        return carry

    lax.fori_loop(0, NPT // WB, wb_block, 0)


def _post_body(p0_ref, p1_ref, x_ref, pe_ref, wmi_ref, wmo_ref, ws_ref,
               bi_ref, bo_ref, cin_ref, cout_ref, wdec_ref, bdec_ref,
               logp_ref, emb_ref):
    f32 = jnp.float32
    h = x_ref[...] + pe_ref[...]
    p = p0_ref[0] + p1_ref[0]
    wa = wmi_ref[...] + ws_ref[...]
    wb = wmo_ref[...] + ws_ref[...]
    dn = (((1,), (1,)), ((), ()))
    ic = lax.dot_general(p, wa, dn, precision=lax.Precision.HIGHEST,
                         preferred_element_type=f32) + bi_ref[...]
    oc = lax.dot_general(p, wb, dn, precision=lax.Precision.HIGHEST,
                         preferred_element_type=f32) + bo_ref[...]
    conv = cin_ref[...] * ic + cout_ref[...] * oc
    h2 = jnp.tanh(conv + h)
    logits = lax.dot_general(h2, wdec_ref[...], dn,
                             precision=lax.Precision.HIGHEST,
                             preferred_element_type=f32) + bdec_ref[...]
    m = jnp.max(logits, axis=1, keepdims=True)
    lse = jnp.log(jnp.sum(jnp.exp(logits - m), axis=1, keepdims=True)) + m
    logp_ref[...] = logits - lse
    nrm = jnp.sqrt(jnp.sum(h2 * h2, axis=1, keepdims=True))
    emb_ref[...] = h2 / (nrm + 1e-12)


def _run_sc_prop(h_pad, src2d, dst2d, zeros):
    f32 = jnp.float32
    sc_prop = functools.partial(
        pl.kernel,
        out_type=jax.ShapeDtypeStruct((NC * NPA, D), f32),
        mesh=plsc.VectorSubcoreMesh(core_axis_name="c", subcore_axis_name="s"),
        scratch_types=[
            pltpu.VMEM((IBR, K), jnp.int32),
            pltpu.VMEM((IBR, K), jnp.int32),
            pltpu.VMEM((NB, K, D), f32),
            pltpu.VMEM_SHARED((NPA, D), f32),
            pltpu.SemaphoreType.DMA((NB,)),
            pltpu.SemaphoreType.DMA((NB,)),
        ],
    )(_sc_prop_body)
    return sc_prop(h_pad, src2d, dst2d, zeros)


_ROWS_BLK = 1000
_GRID = N // _ROWS_BLK


def kernel(x, pe_table, W_main_in, W_main_out, W_shared, b_main_in,
           b_main_out, b_shared_in, b_shared_out, C_in_vec, C_out_vec,
           W_dec, b_dec, edge_index):
    f32 = jnp.float32
    pe_flat = pe_table.reshape(1, D).astype(f32)

    h = pl.pallas_call(
        _pe_body,
        grid=(_GRID,),
        in_specs=[
            pl.BlockSpec((_ROWS_BLK, D), lambda i: (i, 0)),
            pl.BlockSpec((1, D), lambda i: (0, 0)),
        ],
        out_specs=pl.BlockSpec((_ROWS_BLK, D), lambda i: (i, 0)),
        out_shape=jax.ShapeDtypeStruct((N, D), f32),
    )(x, pe_flat)

    src = edge_index[0].astype(jnp.int32)
    dst = edge_index[1].astype(jnp.int32)
    pad = EPC - EH
    padsrc = jnp.full((pad,), N, jnp.int32)
    paddst = jnp.full((pad,), N, jnp.int32)
    src2d = jnp.concatenate([src[:EH], padsrc, src[EH:], padsrc]).reshape(-1, K)
    dst2d = jnp.concatenate([dst[:EH], paddst, dst[EH:], paddst]).reshape(-1, K)
    h_pad = jnp.concatenate([h, jnp.zeros((NTAB - N, D), f32)], axis=0)
    zeros = jnp.zeros((NPA, D), f32)

    p01 = _run_sc_prop(h_pad, src2d, dst2d, zeros).reshape(NC, NPA, D)

    cin = C_in_vec.astype(f32)
    cout = C_out_vec.astype(f32)
    row_spec = pl.BlockSpec((_ROWS_BLK, D), lambda i: (i, 0))
    w_spec = pl.BlockSpec((D, D), lambda i: (0, 0))
    b_spec = pl.BlockSpec((1, D), lambda i: (0, 0))
    c_spec = pl.BlockSpec((_ROWS_BLK, 1), lambda i: (i, 0))
    logp, emb = pl.pallas_call(
        _post_body,
        grid=(_GRID,),
        in_specs=[
            pl.BlockSpec((1, _ROWS_BLK, D), lambda i: (0, i, 0)),
            pl.BlockSpec((1, _ROWS_BLK, D), lambda i: (1, i, 0)),
            row_spec,
            b_spec,
            w_spec, w_spec, w_spec,
            b_spec, b_spec,
            c_spec, c_spec,
            pl.BlockSpec((CLASSES, D), lambda i: (0, 0)),
            pl.BlockSpec((1, CLASSES), lambda i: (0, 0)),
        ],
        out_specs=[
            pl.BlockSpec((_ROWS_BLK, CLASSES), lambda i: (i, 0)),
            pl.BlockSpec((_ROWS_BLK, D), lambda i: (i, 0)),
        ],
        out_shape=[
            jax.ShapeDtypeStruct((N, CLASSES), f32),
            jax.ShapeDtypeStruct((N, D), f32),
        ],
    )(p01, p01, x, pe_flat,
      W_main_in, W_main_out, W_shared,
      (b_main_in + b_shared_in).reshape(1, D),
      (b_main_out + b_shared_out).reshape(1, D),
      cin, cout, W_dec, b_dec.reshape(1, CLASSES))
    return (logp, emb)


# speedup vs baseline: 1.0447x; 1.0144x over previous
import functools

import jax
import jax.numpy as jnp
from jax import lax
from jax.experimental import pallas as pl
from jax.experimental.pallas import tpu as pltpu
from jax.experimental.pallas import tpu_sc as plsc

N = 10000
E = 320000
D = 128
CLASSES = 20

NC = 2
NS = 16
K = 128
NCT = 80
EPC = NCT * K * NS
EH = E // NC
NTAB = N + 8
NPA = 10240
NPT = NPA // NS
WB = 128
NB = 2
IBR = 40
NIB = NCT // IBR


def _pe_body(x_ref, pe_ref, o_ref):
    o_ref[...] = x_ref[...] + pe_ref[...]


def _sc_prop_body(h_hbm, src_hbm, dst_hbm, zeros_hbm, out_hbm,
                  src_b, dst_b, rows_v, acc_sh, gsem, ssem):
    c = lax.axis_index("c")
    s = lax.axis_index("s")
    tile_row0 = (c * NS + s) * NCT
    nrow0 = s * NPT

    pltpu.sync_copy(zeros_hbm.at[pl.ds(nrow0, NPT)], acc_sh.at[pl.ds(nrow0, NPT)])
    plsc.subcore_barrier()

    def fire_g(q, b):
        pltpu.async_copy(h_hbm.at[src_b.at[q]], rows_v.at[b], gsem.at[b])

    def wait_g(q, b):
        pltpu.make_async_copy(h_hbm.at[src_b.at[q]], rows_v.at[b],
                              gsem.at[b]).wait()

    def fire_s(q, b):
        pltpu.async_copy(rows_v.at[b], acc_sh.at[dst_b.at[q]], ssem.at[b],
                         add=True)

    def wait_s(q, b):
        pltpu.make_async_copy(rows_v.at[b], acc_sh.at[dst_b.at[q]],
                              ssem.at[b]).wait()

    def block(ib, carry):
        r0 = tile_row0 + ib * IBR
        pltpu.sync_copy(src_hbm.at[pl.ds(r0, IBR)], src_b)
        pltpu.sync_copy(dst_hbm.at[pl.ds(r0, IBR)], dst_b)
        for b in range(NB):
            fire_g(b, b)

        def pair(g, c2):
            q0 = 2 * g
            for b in range(NB):
                wait_g(q0 + b, b)
                fire_s(q0 + b, b)
                wait_s(q0 + b, b)
                fire_g(q0 + NB + b, b)
            return c2

        lax.fori_loop(0, (IBR - NB) // 2, pair, 0)
        for b in range(NB):
            wait_g(IBR - NB + b, b)
            fire_s(IBR - NB + b, b)
            wait_s(IBR - NB + b, b)
        return carry

    lax.fori_loop(0, NIB, block, 0)
    plsc.subcore_barrier()

    out0 = c * NPA + nrow0

    def wb_block(j, carry):
        pltpu.sync_copy(acc_sh.at[pl.ds(nrow0 + j * WB, WB)], rows_v.at[0])
        pltpu.sync_copy(rows_v.at[0], out_hbm.at[pl.ds(out0 + j * WB, WB)])
        return carry

    lax.fori_loop(0, NPT // WB, wb_block, 0)


def _post_body(p0_ref, p1_ref, x_ref, pe_ref, wmi_ref, wmo_ref, ws_ref,
               bi_ref, bo_ref, cin_ref, cout_ref, wdec_ref, bdec_ref,
               logp_ref, emb_ref):
    f32 = jnp.float32
    h = x_ref[...] + pe_ref[...]
    p = p0_ref[0] + p1_ref[0]
    wa = wmi_ref[...] + ws_ref[...]
    wb = wmo_ref[...] + ws_ref[...]
    dn = (((1,), (1,)), ((), ()))
    ic = lax.dot_general(p, wa, dn, precision=lax.Precision.HIGHEST,
                         preferred_element_type=f32) + bi_ref[...]
    oc = lax.dot_general(p, wb, dn, precision=lax.Precision.HIGHEST,
                         preferred_element_type=f32) + bo_ref[...]
    conv = cin_ref[...] * ic + cout_ref[...] * oc
    h2 = jnp.tanh(conv + h)
    logits = lax.dot_general(h2, wdec_ref[...], dn,
                             precision=lax.Precision.HIGHEST,
                             preferred_element_type=f32) + bdec_ref[...]
    m = jnp.max(logits, axis=1, keepdims=True)
    lse = jnp.log(jnp.sum(jnp.exp(logits - m), axis=1, keepdims=True)) + m
    logp_ref[...] = logits - lse
    nrm = jnp.sqrt(jnp.sum(h2 * h2, axis=1, keepdims=True))
    emb_ref[...] = h2 / (nrm + 1e-12)


def _run_sc_prop(h_pad, src2d, dst2d, zeros):
    f32 = jnp.float32
    sc_prop = functools.partial(
        pl.kernel,
        out_type=jax.ShapeDtypeStruct((NC * NPA, D), f32),
        mesh=plsc.VectorSubcoreMesh(core_axis_name="c", subcore_axis_name="s"),
        scratch_types=[
            pltpu.VMEM((IBR, K), jnp.int32),
            pltpu.VMEM((IBR, K), jnp.int32),
            pltpu.VMEM((NB, K, D), f32),
            pltpu.VMEM_SHARED((NPA, D), f32),
            pltpu.SemaphoreType.DMA((NB,)),
            pltpu.SemaphoreType.DMA((NB,)),
        ],
    )(_sc_prop_body)
    return sc_prop(h_pad, src2d, dst2d, zeros)


_ROWS_BLK = 1000
_GRID = N // _ROWS_BLK


def kernel(x, pe_table, W_main_in, W_main_out, W_shared, b_main_in,
           b_main_out, b_shared_in, b_shared_out, C_in_vec, C_out_vec,
           W_dec, b_dec, edge_index):
    f32 = jnp.float32
    pe_flat = pe_table.reshape(1, D).astype(f32)

    h = pl.pallas_call(
        _pe_body,
        grid=(_GRID,),
        in_specs=[
            pl.BlockSpec((_ROWS_BLK, D), lambda i: (i, 0)),
            pl.BlockSpec((1, D), lambda i: (0, 0)),
        ],
        out_specs=pl.BlockSpec((_ROWS_BLK, D), lambda i: (i, 0)),
        out_shape=jax.ShapeDtypeStruct((N, D), f32),
    )(x, pe_flat)

    src = edge_index[0].astype(jnp.int32)
    dst = edge_index[1].astype(jnp.int32)
    pad = EPC - EH
    padsrc = jnp.full((pad,), N, jnp.int32)
    paddst = jnp.full((pad,), N, jnp.int32)
    src2d = jnp.concatenate([src[:EH], padsrc, src[EH:], padsrc]).reshape(-1, K)
    dst2d = jnp.concatenate([dst[:EH], paddst, dst[EH:], paddst]).reshape(-1, K)
    h_pad = jnp.concatenate([h, jnp.zeros((NTAB - N, D), f32)], axis=0)
    zeros = jnp.zeros((NPA, D), f32)

    p01 = _run_sc_prop(h_pad, src2d, dst2d, zeros).reshape(NC, NPA, D)

    cin = C_in_vec.astype(f32)
    cout = C_out_vec.astype(f32)
    row_spec = pl.BlockSpec((_ROWS_BLK, D), lambda i: (i, 0))
    w_spec = pl.BlockSpec((D, D), lambda i: (0, 0))
    b_spec = pl.BlockSpec((1, D), lambda i: (0, 0))
    c_spec = pl.BlockSpec((_ROWS_BLK, 1), lambda i: (i, 0))
    logp, emb = pl.pallas_call(
        _post_body,
        grid=(_GRID,),
        in_specs=[
            pl.BlockSpec((1, _ROWS_BLK, D), lambda i: (0, i, 0)),
            pl.BlockSpec((1, _ROWS_BLK, D), lambda i: (1, i, 0)),
            row_spec,
            b_spec,
            w_spec, w_spec, w_spec,
            b_spec, b_spec,
            c_spec, c_spec,
            pl.BlockSpec((CLASSES, D), lambda i: (0, 0)),
            pl.BlockSpec((1, CLASSES), lambda i: (0, 0)),
        ],
        out_specs=[
            pl.BlockSpec((_ROWS_BLK, CLASSES), lambda i: (i, 0)),
            pl.BlockSpec((_ROWS_BLK, D), lambda i: (i, 0)),
        ],
        out_shape=[
            jax.ShapeDtypeStruct((N, CLASSES), f32),
            jax.ShapeDtypeStruct((N, D), f32),
        ],
    )(p01, p01, x, pe_flat,
      W_main_in, W_main_out, W_shared,
      (b_main_in + b_shared_in).reshape(1, D),
      (b_main_out + b_shared_out).reshape(1, D),
      cin, cout, W_dec, b_dec.reshape(1, CLASSES))
    return (logp, emb)
